# baseline (speedup 1.0000x reference)
import numpy as np
import jax
import jax.numpy as jnp
from jax import lax
from jax.experimental import pallas as pl
from jax.experimental.pallas import tpu as pltpu
from jax.experimental.pallas import tpu_sc as plsc

B, D, E, KTOP = 2048, 768, 8, 2
T = 256
NT = (B * KTOP + E * T) // T
NPAD = NT * T
BLK = 128
NBLK = B // BLK
MAXKV = B // T
NEG = -1e9


def _routing_body(x_ref, wg_ref, s0_ref, s1_ref, g0_ref, g1_ref, tab_ref):
    logits = jnp.dot(x_ref[...], wg_ref[...],
                     preferred_element_type=jnp.float32)
    iota_e = lax.broadcasted_iota(jnp.int32, (B, E), 1)
    m1 = jnp.max(logits, axis=1, keepdims=True)
    e0 = jnp.min(jnp.where(logits == m1, iota_e, E), axis=1, keepdims=True)
    l2 = jnp.where(iota_e == e0, -jnp.inf, logits)
    m2 = jnp.max(l2, axis=1, keepdims=True)
    e1 = jnp.min(jnp.where(l2 == m2, iota_e, E), axis=1, keepdims=True)
    t = jnp.exp(m2 - m1)
    g0_ref[...] = 1.0 / (1.0 + t)
    g1_ref[...] = 1.0 - 1.0 / (1.0 + t)

    oh0 = (iota_e == e0)
    oh1 = (iota_e == e1)
    mask = (oh0 | oh1).astype(jnp.bfloat16)

    r = lax.broadcasted_iota(jnp.int32, (B, B), 0)
    c = lax.broadcasted_iota(jnp.int32, (B, B), 1)
    tri = (c < r).astype(jnp.bfloat16)
    rank = jnp.dot(tri, mask, preferred_element_type=jnp.float32)
    r0 = jnp.sum(rank * oh0, axis=1, keepdims=True)
    r1 = jnp.sum(rank * oh1, axis=1, keepdims=True)
    cnt = jnp.sum(mask.astype(jnp.float32), axis=0, keepdims=True)

    starts = []
    ntiles = []
    s = jnp.float32(0.0)
    for e in range(E):
        nt_e = jnp.ceil(cnt[0, e] / T)
        starts.append(s)
        ntiles.append(nt_e)
        s = s + nt_e * T
    total_tiles = s / T

    sel0 = jnp.zeros((B, 1), dtype=jnp.float32)
    sel1 = jnp.zeros((B, 1), dtype=jnp.float32)
    for e in range(E):
        sel0 = sel0 + jnp.where(e0 == e, starts[e], 0.0)
        sel1 = sel1 + jnp.where(e1 == e, starts[e], 0.0)
    s0_ref[...] = (sel0 + r0).astype(jnp.int32)
    s1_ref[...] = (sel1 + r1).astype(jnp.int32)

    lane = lax.broadcasted_iota(jnp.int32, (1, 128), 1)
    tab = jnp.zeros((1, 128), jnp.float32)
    for e in range(E):
        st_t = starts[e] / T
        en_t = st_t + ntiles[e]
        texp = jnp.where((lane < NT) & (lane >= st_t) & (lane < en_t),
                         float(e), 0.0)
        tab = tab + texp
        tab = tab + jnp.where(lane == 32 + e, st_t, 0.0)
        tab = tab + jnp.where(lane == 40 + e, ntiles[e], 0.0)
        tab = tab + jnp.where(lane == 48 + e, cnt[0, e], 0.0)
    tab = tab + jnp.where(lane == 120, total_tiles, 0.0)
    tab_ref[...] = tab.astype(jnp.int32)


def _routing(x, w_gate):
    full = pl.BlockSpec((B, 1), lambda: (0, 0))
    return pl.pallas_call(
        _routing_body,
        in_specs=[
            pl.BlockSpec((B, D), lambda: (0, 0)),
            pl.BlockSpec((D, E), lambda: (0, 0)),
        ],
        out_specs=[full, full, full, full,
                   pl.BlockSpec((1, 128), lambda: (0, 0))],
        out_shape=[
            jax.ShapeDtypeStruct((B, 1), jnp.int32),
            jax.ShapeDtypeStruct((B, 1), jnp.int32),
            jax.ShapeDtypeStruct((B, 1), jnp.float32),
            jax.ShapeDtypeStruct((B, 1), jnp.float32),
            jax.ShapeDtypeStruct((1, 128), jnp.int32),
        ],
    )(x, w_gate)


NSC_CORES = 2
NSC_SUB = 16
NWORK = NSC_CORES * NSC_SUB
CHUNK = B // NWORK


def _sc_scatter_body(x_hbm, s0_hbm, s1_hbm, xg_hbm,
                     idx0_v, idx1_v, rows_v, sem0, sem1):
    wid = lax.axis_index("s") * NSC_CORES + lax.axis_index("c")
    base = wid * CHUNK
    pltpu.sync_copy(s0_hbm.at[pl.ds(base, CHUNK)], idx0_v)
    pltpu.sync_copy(s1_hbm.at[pl.ds(base, CHUNK)], idx1_v)
    pltpu.sync_copy(x_hbm.at[pl.ds(base, CHUNK)], rows_v)
    c0 = pltpu.async_copy(rows_v, xg_hbm.at[idx0_v], sem0)
    c1 = pltpu.async_copy(rows_v, xg_hbm.at[idx1_v], sem1)
    c0.wait()
    c1.wait()


def _sc_scatter(x, s0, s1):
    mesh = plsc.VectorSubcoreMesh(core_axis_name="c", subcore_axis_name="s")
    return pl.kernel(
        _sc_scatter_body,
        out_type=jax.ShapeDtypeStruct((NPAD, D), jnp.float32),
        mesh=mesh,
        scratch_types=[
            pltpu.VMEM((CHUNK,), jnp.int32),
            pltpu.VMEM((CHUNK,), jnp.int32),
            pltpu.VMEM((CHUNK, D), jnp.float32),
            pltpu.SemaphoreType.DMA,
            pltpu.SemaphoreType.DMA,
        ],
    )(x, s0, s1)


def _sc_gather_body(cg_hbm, s0_hbm, s1_hbm, c0_hbm, c1_hbm,
                    idx0_v, idx1_v, rows0_v, rows1_v, sem0, sem1):
    wid = lax.axis_index("s") * NSC_CORES + lax.axis_index("c")
    base = wid * CHUNK
    pltpu.sync_copy(s0_hbm.at[pl.ds(base, CHUNK)], idx0_v)
    pltpu.sync_copy(s1_hbm.at[pl.ds(base, CHUNK)], idx1_v)
    g0 = pltpu.async_copy(cg_hbm.at[idx0_v], rows0_v, sem0)
    g1 = pltpu.async_copy(cg_hbm.at[idx1_v], rows1_v, sem1)
    g0.wait()
    pltpu.sync_copy(rows0_v, c0_hbm.at[pl.ds(base, CHUNK)])
    g1.wait()
    pltpu.sync_copy(rows1_v, c1_hbm.at[pl.ds(base, CHUNK)])


def _sc_gather(cg, s0, s1):
    mesh = plsc.VectorSubcoreMesh(core_axis_name="c", subcore_axis_name="s")
    return pl.kernel(
        _sc_gather_body,
        out_type=[jax.ShapeDtypeStruct((B, D), jnp.float32),
                  jax.ShapeDtypeStruct((B, D), jnp.float32)],
        mesh=mesh,
        scratch_types=[
            pltpu.VMEM((CHUNK,), jnp.int32),
            pltpu.VMEM((CHUNK,), jnp.int32),
            pltpu.VMEM((CHUNK, D), jnp.float32),
            pltpu.VMEM((CHUNK, D), jnp.float32),
            pltpu.SemaphoreType.DMA,
            pltpu.SemaphoreType.DMA,
        ],
    )(cg, s0, s1)


def _mega_body(tab_ref, xg_ref, wq_ref, wk_ref, wv_ref, wo_ref, cg_ref,
               qs_ref, ks_ref, vs_ref, s_ref, acc_ref,
               wqb_ref, wkb_ref, wvb_ref, wob_ref):
    i = pl.program_id(0)
    scale = np.float32(1.0 / np.sqrt(np.float32(D)))
    tot = tab_ref[120]

    @pl.when(i < tot)
    def _():
        e = tab_ref[i]
        nvalid = tab_ref[48 + e] - (i - tab_ref[32 + e]) * T

        @pl.when((i == 0) | (e != tab_ref[jnp.maximum(i - 1, 0)]))
        def _():
            wqb_ref[...] = wq_ref[0].astype(jnp.bfloat16)
            wkb_ref[...] = wk_ref[0].astype(jnp.bfloat16)
            wvb_ref[...] = wv_ref[0].astype(jnp.bfloat16)

        xt = xg_ref[...].astype(jnp.bfloat16)
        q = jnp.dot(xt, wqb_ref[...], preferred_element_type=jnp.float32)
        k = jnp.dot(xt, wkb_ref[...], preferred_element_type=jnp.float32)
        v = jnp.dot(xt, wvb_ref[...], preferred_element_type=jnp.float32)
        sl = pl.ds(i * T, T)
        qs_ref[sl, :] = q.astype(jnp.bfloat16)
        ks_ref[sl, :] = k.astype(jnp.bfloat16)

        @pl.when(nvalid >= T)
        def _():
            vs_ref[sl, :] = v.astype(jnp.bfloat16)

        @pl.when(nvalid < T)
        def _():
            rows = lax.broadcasted_iota(jnp.int32, (T, 1), 0)
            vs_ref[sl, :] = jnp.where(rows < nvalid, v, 0.0).astype(
                jnp.bfloat16)

    t = i - NT

    @pl.when((i >= NT) & (t < tot))
    def _():
        e = tab_ref[t]
        ntile = tab_ref[40 + e]
        n_e = tab_ref[48 + e]
        st = tab_ref[32 + e]

        @pl.when((t == 0) | (e != tab_ref[jnp.maximum(t - 1, 0)]))
        def _():
            wob_ref[...] = wo_ref[0].astype(jnp.bfloat16)

        q = qs_ref[pl.ds(t * T, T), :]

        def body1(j, m):
            kt = ks_ref[pl.ds((st + j) * T, T), :]
            s = jax.lax.dot_general(
                q, kt, (((1,), (1,)), ((), ())),
                preferred_element_type=jnp.float32) * scale
            kcol = lax.broadcasted_iota(jnp.int32, (T, T), 1) + j * T
            s = jnp.where(kcol < n_e, s, NEG)
            s_ref[:, pl.ds(j * T, T)] = s
            return jnp.maximum(m, jnp.max(s, axis=1, keepdims=True))

        m = lax.fori_loop(0, ntile, body1,
                          jnp.full((T, 1), -jnp.inf, jnp.float32))

        acc_ref[...] = jnp.zeros_like(acc_ref)

        def body2(j, l):
            p = jnp.exp(s_ref[:, pl.ds(j * T, T)] - m)
            vt = vs_ref[pl.ds((st + j) * T, T), :]
            acc_ref[...] += jnp.dot(p.astype(jnp.bfloat16), vt,
                                    preferred_element_type=jnp.float32)
            return l + jnp.sum(p, axis=1, keepdims=True)

        l = lax.fori_loop(0, ntile, body2, jnp.zeros((T, 1), jnp.float32))

        o = (acc_ref[...] / l).astype(jnp.bfloat16)
        og = jnp.dot(o, wob_ref[...], preferred_element_type=jnp.float32)
        cg_ref[...] = jnp.exp(og)


def _mega(tab, xg, Wq, Wk, Wv, Wo):
    tile_a = lambda i, tab: (jnp.minimum(i, NT - 1), 0)
    w_a = lambda i, tab: (tab[jnp.minimum(i, NT - 1)], 0, 0)
    w_b = lambda i, tab: (tab[jnp.maximum(i - NT, 0)], 0, 0)
    tile_b = lambda i, tab: (jnp.maximum(i - NT, 0), 0)
    return pl.pallas_call(
        _mega_body,
        grid_spec=pltpu.PrefetchScalarGridSpec(
            num_scalar_prefetch=1,
            grid=(2 * NT,),
            in_specs=[
                pl.BlockSpec((T, D), tile_a),
                pl.BlockSpec((1, D, D), w_a),
                pl.BlockSpec((1, D, D), w_a),
                pl.BlockSpec((1, D, D), w_a),
                pl.BlockSpec((1, D, D), w_b),
            ],
            out_specs=pl.BlockSpec((T, D), tile_b),
            scratch_shapes=[
                pltpu.VMEM((NPAD, D), jnp.bfloat16),
                pltpu.VMEM((NPAD, D), jnp.bfloat16),
                pltpu.VMEM((NPAD, D), jnp.bfloat16),
                pltpu.VMEM((T, MAXKV * T), jnp.float32),
                pltpu.VMEM((T, D), jnp.float32),
                pltpu.VMEM((D, D), jnp.bfloat16),
                pltpu.VMEM((D, D), jnp.bfloat16),
                pltpu.VMEM((D, D), jnp.bfloat16),
                pltpu.VMEM((D, D), jnp.bfloat16),
            ],
        ),
        out_shape=jax.ShapeDtypeStruct((NPAD, D), jnp.float32),
    )(tab, xg, Wq, Wk, Wv, Wo)


def _combine_body(c0_ref, c1_ref, g0_ref, g1_ref, y_ref):
    comb = g0_ref[...] * c0_ref[...] + g1_ref[...] * c1_ref[...]
    eps = np.float32(np.finfo(np.float64).eps)
    comb = jnp.where(comb == 0.0, eps, comb)
    y_ref[...] = jnp.log(comb)


def _combine(c0, c1, g0, g1):
    cblk = 512
    row = pl.BlockSpec((cblk, D), lambda i: (i, 0))
    gsp = pl.BlockSpec((cblk, 1), lambda i: (i, 0))
    return pl.pallas_call(
        _combine_body,
        grid=(B // cblk,),
        in_specs=[row, row, gsp, gsp],
        out_specs=row,
        out_shape=jax.ShapeDtypeStruct((B, D), jnp.float32),
    )(c0, c1, g0, g1)


@jax.jit
def kernel(x, w_gate, Wq, Wk, Wv, Wo):
    s0, s1, g0, g1, tab = _routing(x, w_gate)
    tab1d = tab.reshape(128)
    s0f = s0.reshape(B)
    s1f = s1.reshape(B)
    xg = _sc_scatter(x, s0f, s1f)
    cg = _mega(tab1d, xg, Wq, Wk, Wv, Wo)
    c0, c1 = _sc_gather(cg, s0f, s1f)
    return _combine(c0, c1, g0, g1)


# speedup vs baseline: 1.0978x; 1.0978x over previous
import numpy as np
import jax
import jax.numpy as jnp
from jax import lax
from jax.experimental import pallas as pl
from jax.experimental.pallas import tpu as pltpu
from jax.experimental.pallas import tpu_sc as plsc

B, D, E, KTOP = 2048, 768, 8, 2
T = 256
NT = (B * KTOP + E * T) // T
NPAD = NT * T
BLK = 128
NBLK = B // BLK
MAXKV = B // T
NEG = -1e9


def _routing_body(x_ref, wg_ref, s0_ref, s1_ref, g0_ref, g1_ref, tab_ref):
    logits = jnp.dot(x_ref[...], wg_ref[...],
                     preferred_element_type=jnp.float32)
    iota_e = lax.broadcasted_iota(jnp.int32, (B, E), 1)
    m1 = jnp.max(logits, axis=1, keepdims=True)
    e0 = jnp.min(jnp.where(logits == m1, iota_e, E), axis=1, keepdims=True)
    l2 = jnp.where(iota_e == e0, -jnp.inf, logits)
    m2 = jnp.max(l2, axis=1, keepdims=True)
    e1 = jnp.min(jnp.where(l2 == m2, iota_e, E), axis=1, keepdims=True)
    t = jnp.exp(m2 - m1)
    g0_ref[...] = 1.0 / (1.0 + t)
    g1_ref[...] = 1.0 - 1.0 / (1.0 + t)

    oh0 = (iota_e == e0)
    oh1 = (iota_e == e1)
    mask = (oh0 | oh1).astype(jnp.bfloat16)

    r = lax.broadcasted_iota(jnp.int32, (B, B), 0)
    c = lax.broadcasted_iota(jnp.int32, (B, B), 1)
    tri = (c < r).astype(jnp.bfloat16)
    rank = jnp.dot(tri, mask, preferred_element_type=jnp.float32)
    r0 = jnp.sum(rank * oh0, axis=1, keepdims=True)
    r1 = jnp.sum(rank * oh1, axis=1, keepdims=True)
    cnt = jnp.sum(mask.astype(jnp.float32), axis=0, keepdims=True)

    starts = []
    ntiles = []
    s = jnp.float32(0.0)
    for e in range(E):
        nt_e = jnp.ceil(cnt[0, e] / T)
        starts.append(s)
        ntiles.append(nt_e)
        s = s + nt_e * T
    total_tiles = s / T

    sel0 = jnp.zeros((B, 1), dtype=jnp.float32)
    sel1 = jnp.zeros((B, 1), dtype=jnp.float32)
    for e in range(E):
        sel0 = sel0 + jnp.where(e0 == e, starts[e], 0.0)
        sel1 = sel1 + jnp.where(e1 == e, starts[e], 0.0)
    s0_ref[...] = (sel0 + r0).astype(jnp.int32)
    s1_ref[...] = (sel1 + r1).astype(jnp.int32)

    lane = lax.broadcasted_iota(jnp.int32, (1, 128), 1)
    tab = jnp.zeros((1, 128), jnp.float32)
    for e in range(E):
        st_t = starts[e] / T
        en_t = st_t + ntiles[e]
        texp = jnp.where((lane < NT) & (lane >= st_t) & (lane < en_t),
                         float(e), 0.0)
        tab = tab + texp
        tab = tab + jnp.where(lane == 32 + e, st_t, 0.0)
        tab = tab + jnp.where(lane == 40 + e, ntiles[e], 0.0)
        tab = tab + jnp.where(lane == 48 + e, cnt[0, e], 0.0)
    tab = tab + jnp.where(lane == 120, total_tiles, 0.0)
    tab_ref[...] = tab.astype(jnp.int32)


def _routing(x, w_gate):
    full = pl.BlockSpec((B, 1), lambda: (0, 0))
    return pl.pallas_call(
        _routing_body,
        in_specs=[
            pl.BlockSpec((B, D), lambda: (0, 0)),
            pl.BlockSpec((D, E), lambda: (0, 0)),
        ],
        out_specs=[full, full, full, full,
                   pl.BlockSpec((1, 128), lambda: (0, 0))],
        out_shape=[
            jax.ShapeDtypeStruct((B, 1), jnp.int32),
            jax.ShapeDtypeStruct((B, 1), jnp.int32),
            jax.ShapeDtypeStruct((B, 1), jnp.float32),
            jax.ShapeDtypeStruct((B, 1), jnp.float32),
            jax.ShapeDtypeStruct((1, 128), jnp.int32),
        ],
    )(x, w_gate)


NSC_CORES = 2
NSC_SUB = 16
NWORK = NSC_CORES * NSC_SUB
CHUNK = B // NWORK


def _sc_scatter_body(x_hbm, s0_hbm, s1_hbm, xg_hbm,
                     idx0_v, idx1_v, rows_v, sem0, sem1):
    wid = lax.axis_index("s") * NSC_CORES + lax.axis_index("c")
    base = wid * CHUNK
    pltpu.sync_copy(s0_hbm.at[pl.ds(base, CHUNK)], idx0_v)
    pltpu.sync_copy(s1_hbm.at[pl.ds(base, CHUNK)], idx1_v)
    pltpu.sync_copy(x_hbm.at[pl.ds(base, CHUNK)], rows_v)
    c0 = pltpu.async_copy(rows_v, xg_hbm.at[idx0_v], sem0)
    c1 = pltpu.async_copy(rows_v, xg_hbm.at[idx1_v], sem1)
    c0.wait()
    c1.wait()


def _sc_scatter(x, s0, s1):
    mesh = plsc.VectorSubcoreMesh(core_axis_name="c", subcore_axis_name="s")
    return pl.kernel(
        _sc_scatter_body,
        out_type=jax.ShapeDtypeStruct((NPAD, D), jnp.float32),
        mesh=mesh,
        scratch_types=[
            pltpu.VMEM((CHUNK,), jnp.int32),
            pltpu.VMEM((CHUNK,), jnp.int32),
            pltpu.VMEM((CHUNK, D), jnp.float32),
            pltpu.SemaphoreType.DMA,
            pltpu.SemaphoreType.DMA,
        ],
    )(x, s0, s1)


def _sc_gather_body(cg_hbm, s0_hbm, s1_hbm, c0_hbm, c1_hbm,
                    idx0_v, idx1_v, rows0_v, rows1_v, sem0, sem1):
    wid = lax.axis_index("s") * NSC_CORES + lax.axis_index("c")
    base = wid * CHUNK
    pltpu.sync_copy(s0_hbm.at[pl.ds(base, CHUNK)], idx0_v)
    pltpu.sync_copy(s1_hbm.at[pl.ds(base, CHUNK)], idx1_v)
    g0 = pltpu.async_copy(cg_hbm.at[idx0_v], rows0_v, sem0)
    g1 = pltpu.async_copy(cg_hbm.at[idx1_v], rows1_v, sem1)
    g0.wait()
    pltpu.sync_copy(rows0_v, c0_hbm.at[pl.ds(base, CHUNK)])
    g1.wait()
    pltpu.sync_copy(rows1_v, c1_hbm.at[pl.ds(base, CHUNK)])


def _sc_gather(cg, s0, s1):
    mesh = plsc.VectorSubcoreMesh(core_axis_name="c", subcore_axis_name="s")
    return pl.kernel(
        _sc_gather_body,
        out_type=[jax.ShapeDtypeStruct((B, D), jnp.float32),
                  jax.ShapeDtypeStruct((B, D), jnp.float32)],
        mesh=mesh,
        scratch_types=[
            pltpu.VMEM((CHUNK,), jnp.int32),
            pltpu.VMEM((CHUNK,), jnp.int32),
            pltpu.VMEM((CHUNK, D), jnp.float32),
            pltpu.VMEM((CHUNK, D), jnp.float32),
            pltpu.SemaphoreType.DMA,
            pltpu.SemaphoreType.DMA,
        ],
    )(cg, s0, s1)


def _mega_body(tab_ref, xg_ref, wq_ref, wk_ref, wv_ref, wo_ref, cg_ref,
               qs_ref, ks_ref, vs_ref, s_ref, acc_ref):
    i = pl.program_id(0)
    scale = np.float32(1.0 / np.sqrt(np.float32(D)))
    tot = tab_ref[120]

    @pl.when(i < tot)
    def _():
        e = tab_ref[i]
        nvalid = tab_ref[48 + e] - (i - tab_ref[32 + e]) * T
        xt = xg_ref[...].astype(jnp.bfloat16)
        q = jnp.dot(xt, wq_ref[0].astype(jnp.bfloat16),
                    preferred_element_type=jnp.float32)
        k = jnp.dot(xt, wk_ref[0].astype(jnp.bfloat16),
                    preferred_element_type=jnp.float32)
        v = jnp.dot(xt, wv_ref[0].astype(jnp.bfloat16),
                    preferred_element_type=jnp.float32)
        sl = pl.ds(i * T, T)
        qs_ref[sl, :] = q.astype(jnp.bfloat16)
        ks_ref[sl, :] = k.astype(jnp.bfloat16)

        @pl.when(nvalid >= T)
        def _():
            vs_ref[sl, :] = v.astype(jnp.bfloat16)

        @pl.when(nvalid < T)
        def _():
            rows = lax.broadcasted_iota(jnp.int32, (T, 1), 0)
            vs_ref[sl, :] = jnp.where(rows < nvalid, v, 0.0).astype(
                jnp.bfloat16)

    t = i - MAXKV

    @pl.when((i >= MAXKV) & (t < tot))
    def _():
        e = tab_ref[t]
        ntile = tab_ref[40 + e]
        n_e = tab_ref[48 + e]
        st = tab_ref[32 + e]
        q = qs_ref[pl.ds(t * T, T), :]

        def body1(j, m):
            kt = ks_ref[pl.ds((st + j) * T, T), :]
            s = jax.lax.dot_general(
                q, kt, (((1,), (1,)), ((), ())),
                preferred_element_type=jnp.float32) * scale
            kcol = lax.broadcasted_iota(jnp.int32, (T, T), 1) + j * T
            s = jnp.where(kcol < n_e, s, NEG)
            s_ref[:, pl.ds(j * T, T)] = s
            return jnp.maximum(m, jnp.max(s, axis=1, keepdims=True))

        m = lax.fori_loop(0, ntile, body1,
                          jnp.full((T, 1), -jnp.inf, jnp.float32))

        acc_ref[...] = jnp.zeros_like(acc_ref)

        def body2(j, l):
            p = jnp.exp(s_ref[:, pl.ds(j * T, T)] - m)
            vt = vs_ref[pl.ds((st + j) * T, T), :]
            acc_ref[...] += jnp.dot(p.astype(jnp.bfloat16), vt,
                                    preferred_element_type=jnp.float32)
            return l + jnp.sum(p, axis=1, keepdims=True)

        l = lax.fori_loop(0, ntile, body2, jnp.zeros((T, 1), jnp.float32))

        o = (acc_ref[...] / l).astype(jnp.bfloat16)
        og = jnp.dot(o, wo_ref[0].astype(jnp.bfloat16),
                     preferred_element_type=jnp.float32)
        cg_ref[...] = jnp.exp(og)


def _mega(tab, xg, Wq, Wk, Wv, Wo):
    tile_a = lambda i, tab: (jnp.minimum(i, NT - 1), 0)
    w_a = lambda i, tab: (tab[jnp.minimum(i, NT - 1)], 0, 0)
    w_b = lambda i, tab: (tab[jnp.maximum(i - MAXKV, 0)], 0, 0)
    tile_b = lambda i, tab: (jnp.maximum(i - MAXKV, 0), 0)
    return pl.pallas_call(
        _mega_body,
        grid_spec=pltpu.PrefetchScalarGridSpec(
            num_scalar_prefetch=1,
            grid=(NT + MAXKV,),
            in_specs=[
                pl.BlockSpec((T, D), tile_a),
                pl.BlockSpec((1, D, D), w_a),
                pl.BlockSpec((1, D, D), w_a),
                pl.BlockSpec((1, D, D), w_a),
                pl.BlockSpec((1, D, D), w_b),
            ],
            out_specs=pl.BlockSpec((T, D), tile_b),
            scratch_shapes=[
                pltpu.VMEM((NPAD, D), jnp.bfloat16),
                pltpu.VMEM((NPAD, D), jnp.bfloat16),
                pltpu.VMEM((NPAD, D), jnp.bfloat16),
                pltpu.VMEM((T, MAXKV * T), jnp.float32),
                pltpu.VMEM((T, D), jnp.float32),
            ],
        ),
        out_shape=jax.ShapeDtypeStruct((NPAD, D), jnp.float32),
    )(tab, xg, Wq, Wk, Wv, Wo)


def _combine_body(c0_ref, c1_ref, g0_ref, g1_ref, y_ref):
    comb = g0_ref[...] * c0_ref[...] + g1_ref[...] * c1_ref[...]
    eps = np.float32(np.finfo(np.float64).eps)
    comb = jnp.where(comb == 0.0, eps, comb)
    y_ref[...] = jnp.log(comb)


def _combine(c0, c1, g0, g1):
    cblk = 512
    row = pl.BlockSpec((cblk, D), lambda i: (i, 0))
    gsp = pl.BlockSpec((cblk, 1), lambda i: (i, 0))
    return pl.pallas_call(
        _combine_body,
        grid=(B // cblk,),
        in_specs=[row, row, gsp, gsp],
        out_specs=row,
        out_shape=jax.ShapeDtypeStruct((B, D), jnp.float32),
    )(c0, c1, g0, g1)


@jax.jit
def kernel(x, w_gate, Wq, Wk, Wv, Wo):
    s0, s1, g0, g1, tab = _routing(x, w_gate)
    tab1d = tab.reshape(128)
    s0f = s0.reshape(B)
    s1f = s1.reshape(B)
    xg = _sc_scatter(x, s0f, s1f)
    cg = _mega(tab1d, xg, Wq, Wk, Wv, Wo)
    c0, c1 = _sc_gather(cg, s0f, s1f)
    return _combine(c0, c1, g0, g1)


# speedup vs baseline: 1.1152x; 1.0158x over previous
import numpy as np
import jax
import jax.numpy as jnp
from jax import lax
from jax.experimental import pallas as pl
from jax.experimental.pallas import tpu as pltpu
from jax.experimental.pallas import tpu_sc as plsc

B, D, E, KTOP = 2048, 768, 8, 2
T = 256
NT = (B * KTOP + E * T) // T
NPAD = NT * T
BLK = 128
NBLK = B // BLK
MAXKV = B // T
NEG = -1e9


def _routing_body(x_ref, wg_ref, s0_ref, s1_ref, g0_ref, g1_ref, tab_ref):
    logits = jnp.dot(x_ref[...], wg_ref[...],
                     preferred_element_type=jnp.float32)
    iota_e = lax.broadcasted_iota(jnp.int32, (B, E), 1)
    m1 = jnp.max(logits, axis=1, keepdims=True)
    e0 = jnp.min(jnp.where(logits == m1, iota_e, E), axis=1, keepdims=True)
    l2 = jnp.where(iota_e == e0, -jnp.inf, logits)
    m2 = jnp.max(l2, axis=1, keepdims=True)
    e1 = jnp.min(jnp.where(l2 == m2, iota_e, E), axis=1, keepdims=True)
    t = jnp.exp(m2 - m1)
    g0_ref[...] = 1.0 / (1.0 + t)
    g1_ref[...] = 1.0 - 1.0 / (1.0 + t)

    oh0 = (iota_e == e0)
    oh1 = (iota_e == e1)
    mask = (oh0 | oh1).astype(jnp.bfloat16)

    r = lax.broadcasted_iota(jnp.int32, (B, B), 0)
    c = lax.broadcasted_iota(jnp.int32, (B, B), 1)
    tri = (c < r).astype(jnp.bfloat16)
    rank = jnp.dot(tri, mask, preferred_element_type=jnp.float32)
    r0 = jnp.sum(rank * oh0, axis=1, keepdims=True)
    r1 = jnp.sum(rank * oh1, axis=1, keepdims=True)
    cnt = jnp.sum(mask.astype(jnp.float32), axis=0, keepdims=True)

    starts = []
    ntiles = []
    s = jnp.float32(0.0)
    for e in range(E):
        nt_e = jnp.ceil(cnt[0, e] / T)
        starts.append(s)
        ntiles.append(nt_e)
        s = s + nt_e * T
    total_tiles = s / T

    sel0 = jnp.zeros((B, 1), dtype=jnp.float32)
    sel1 = jnp.zeros((B, 1), dtype=jnp.float32)
    for e in range(E):
        sel0 = sel0 + jnp.where(e0 == e, starts[e], 0.0)
        sel1 = sel1 + jnp.where(e1 == e, starts[e], 0.0)
    s0_ref[...] = (sel0 + r0).astype(jnp.int32)
    s1_ref[...] = (sel1 + r1).astype(jnp.int32)

    lane = lax.broadcasted_iota(jnp.int32, (1, 128), 1)
    tab = jnp.zeros((1, 128), jnp.float32)
    for e in range(E):
        st_t = starts[e] / T
        en_t = st_t + ntiles[e]
        texp = jnp.where((lane < NT) & (lane >= st_t) & (lane < en_t),
                         float(e), 0.0)
        tab = tab + texp
        tab = tab + jnp.where(lane == 32 + e, st_t, 0.0)
        tab = tab + jnp.where(lane == 40 + e, ntiles[e], 0.0)
        tab = tab + jnp.where(lane == 48 + e, cnt[0, e], 0.0)
    tab = tab + jnp.where(lane == 120, total_tiles, 0.0)
    tab_ref[...] = tab.astype(jnp.int32)


def _routing(x, w_gate):
    full = pl.BlockSpec((B, 1), lambda: (0, 0))
    return pl.pallas_call(
        _routing_body,
        in_specs=[
            pl.BlockSpec((B, D), lambda: (0, 0)),
            pl.BlockSpec((D, E), lambda: (0, 0)),
        ],
        out_specs=[full, full, full, full,
                   pl.BlockSpec((1, 128), lambda: (0, 0))],
        out_shape=[
            jax.ShapeDtypeStruct((B, 1), jnp.int32),
            jax.ShapeDtypeStruct((B, 1), jnp.int32),
            jax.ShapeDtypeStruct((B, 1), jnp.float32),
            jax.ShapeDtypeStruct((B, 1), jnp.float32),
            jax.ShapeDtypeStruct((1, 128), jnp.int32),
        ],
    )(x, w_gate)


NSC_CORES = 2
NSC_SUB = 16
NWORK = NSC_CORES * NSC_SUB
CHUNK = B // NWORK


def _sc_scatter_body(x_hbm, s0_hbm, s1_hbm, xg_hbm,
                     idx0_v, idx1_v, rows_v, sem0, sem1):
    wid = lax.axis_index("s") * NSC_CORES + lax.axis_index("c")
    base = wid * CHUNK
    i0 = pltpu.async_copy(s0_hbm.at[pl.ds(base, CHUNK)], idx0_v, sem0)
    i1 = pltpu.async_copy(s1_hbm.at[pl.ds(base, CHUNK)], idx1_v, sem1)
    pltpu.sync_copy(x_hbm.at[pl.ds(base, CHUNK)], rows_v)
    i0.wait()
    i1.wait()
    c0 = pltpu.async_copy(rows_v, xg_hbm.at[idx0_v], sem0)
    c1 = pltpu.async_copy(rows_v, xg_hbm.at[idx1_v], sem1)
    c0.wait()
    c1.wait()


def _sc_scatter(x, s0, s1):
    mesh = plsc.VectorSubcoreMesh(core_axis_name="c", subcore_axis_name="s")
    return pl.kernel(
        _sc_scatter_body,
        out_type=jax.ShapeDtypeStruct((NPAD, D), jnp.float32),
        mesh=mesh,
        scratch_types=[
            pltpu.VMEM((CHUNK,), jnp.int32),
            pltpu.VMEM((CHUNK,), jnp.int32),
            pltpu.VMEM((CHUNK, D), jnp.float32),
            pltpu.SemaphoreType.DMA,
            pltpu.SemaphoreType.DMA,
        ],
    )(x, s0, s1)


def _sc_gather_body(cg_hbm, s0_hbm, s1_hbm, c0_hbm, c1_hbm,
                    idx0_v, idx1_v, rows0_v, rows1_v, sem0, sem1):
    wid = lax.axis_index("s") * NSC_CORES + lax.axis_index("c")
    base = wid * CHUNK
    pltpu.sync_copy(s0_hbm.at[pl.ds(base, CHUNK)], idx0_v)
    pltpu.sync_copy(s1_hbm.at[pl.ds(base, CHUNK)], idx1_v)
    g0 = pltpu.async_copy(cg_hbm.at[idx0_v], rows0_v, sem0)
    g1 = pltpu.async_copy(cg_hbm.at[idx1_v], rows1_v, sem1)
    g0.wait()
    pltpu.sync_copy(rows0_v, c0_hbm.at[pl.ds(base, CHUNK)])
    g1.wait()
    pltpu.sync_copy(rows1_v, c1_hbm.at[pl.ds(base, CHUNK)])


def _sc_gather(cg, s0, s1):
    mesh = plsc.VectorSubcoreMesh(core_axis_name="c", subcore_axis_name="s")
    return pl.kernel(
        _sc_gather_body,
        out_type=[jax.ShapeDtypeStruct((B, D), jnp.float32),
                  jax.ShapeDtypeStruct((B, D), jnp.float32)],
        mesh=mesh,
        scratch_types=[
            pltpu.VMEM((CHUNK,), jnp.int32),
            pltpu.VMEM((CHUNK,), jnp.int32),
            pltpu.VMEM((CHUNK, D), jnp.float32),
            pltpu.VMEM((CHUNK, D), jnp.float32),
            pltpu.SemaphoreType.DMA,
            pltpu.SemaphoreType.DMA,
        ],
    )(cg, s0, s1)


def _mega_body(tab_ref, xg_ref, wq_ref, wk_ref, wv_ref, wo_ref, cg_ref,
               qs_ref, ks_ref, vs_ref, s_ref, acc_ref):
    i = pl.program_id(0)
    scale = np.float32(1.0 / np.sqrt(np.float32(D)))
    tot = tab_ref[120]

    @pl.when(i < tot)
    def _():
        e = tab_ref[i]
        nvalid = tab_ref[48 + e] - (i - tab_ref[32 + e]) * T
        xt = xg_ref[...].astype(jnp.bfloat16)
        q = jnp.dot(xt, wq_ref[0].astype(jnp.bfloat16),
                    preferred_element_type=jnp.float32)
        k = jnp.dot(xt, wk_ref[0].astype(jnp.bfloat16),
                    preferred_element_type=jnp.float32)
        v = jnp.dot(xt, wv_ref[0].astype(jnp.bfloat16),
                    preferred_element_type=jnp.float32)
        sl = pl.ds(i * T, T)
        qs_ref[sl, :] = q.astype(jnp.bfloat16)
        ks_ref[sl, :] = k.astype(jnp.bfloat16)

        @pl.when(nvalid >= T)
        def _():
            vs_ref[sl, :] = v.astype(jnp.bfloat16)

        @pl.when(nvalid < T)
        def _():
            rows = lax.broadcasted_iota(jnp.int32, (T, 1), 0)
            vs_ref[sl, :] = jnp.where(rows < nvalid, v, 0.0).astype(
                jnp.bfloat16)

    t = i - MAXKV

    @pl.when((i >= MAXKV) & (t < tot))
    def _():
        e = tab_ref[t]
        ntile = tab_ref[40 + e]
        n_e = tab_ref[48 + e]
        st = tab_ref[32 + e]
        q = qs_ref[pl.ds(t * T, T), :]

        def body1(j, m):
            kt = ks_ref[pl.ds((st + j) * T, T), :]
            s = jax.lax.dot_general(
                q, kt, (((1,), (1,)), ((), ())),
                preferred_element_type=jnp.float32) * scale
            s_ref[:, pl.ds(j * T, T)] = s
            return jnp.maximum(m, jnp.max(s, axis=1, keepdims=True))

        m = lax.fori_loop(0, ntile - 1, body1,
                          jnp.full((T, 1), -jnp.inf, jnp.float32))
        jl = ntile - 1
        kt = ks_ref[pl.ds((st + jl) * T, T), :]
        s = jax.lax.dot_general(
            q, kt, (((1,), (1,)), ((), ())),
            preferred_element_type=jnp.float32) * scale
        kcol = lax.broadcasted_iota(jnp.int32, (T, T), 1) + jl * T
        s = jnp.where(kcol < n_e, s, NEG)
        s_ref[:, pl.ds(jl * T, T)] = s
        m = jnp.maximum(m, jnp.max(s, axis=1, keepdims=True))

        acc_ref[...] = jnp.zeros_like(acc_ref)

        def body2(j, l):
            p = jnp.exp(s_ref[:, pl.ds(j * T, T)] - m)
            vt = vs_ref[pl.ds((st + j) * T, T), :]
            acc_ref[...] += jnp.dot(p.astype(jnp.bfloat16), vt,
                                    preferred_element_type=jnp.float32)
            return l + jnp.sum(p, axis=1, keepdims=True)

        l = lax.fori_loop(0, ntile, body2, jnp.zeros((T, 1), jnp.float32))

        o = (acc_ref[...] / l).astype(jnp.bfloat16)
        og = jnp.dot(o, wo_ref[0].astype(jnp.bfloat16),
                     preferred_element_type=jnp.float32)
        cg_ref[...] = jnp.exp(og)


def _mega(tab, xg, Wq, Wk, Wv, Wo):
    tile_a = lambda i, tab: (jnp.minimum(i, NT - 1), 0)
    w_a = lambda i, tab: (tab[jnp.minimum(i, NT - 1)], 0, 0)
    w_b = lambda i, tab: (tab[jnp.maximum(i - MAXKV, 0)], 0, 0)
    tile_b = lambda i, tab: (jnp.maximum(i - MAXKV, 0), 0)
    return pl.pallas_call(
        _mega_body,
        grid_spec=pltpu.PrefetchScalarGridSpec(
            num_scalar_prefetch=1,
            grid=(NT + MAXKV,),
            in_specs=[
                pl.BlockSpec((T, D), tile_a),
                pl.BlockSpec((1, D, D), w_a),
                pl.BlockSpec((1, D, D), w_a),
                pl.BlockSpec((1, D, D), w_a),
                pl.BlockSpec((1, D, D), w_b),
            ],
            out_specs=pl.BlockSpec((T, D), tile_b),
            scratch_shapes=[
                pltpu.VMEM((NPAD, D), jnp.bfloat16),
                pltpu.VMEM((NPAD, D), jnp.bfloat16),
                pltpu.VMEM((NPAD, D), jnp.bfloat16),
                pltpu.VMEM((T, MAXKV * T), jnp.float32),
                pltpu.VMEM((T, D), jnp.float32),
            ],
        ),
        out_shape=jax.ShapeDtypeStruct((NPAD, D), jnp.float32),
    )(tab, xg, Wq, Wk, Wv, Wo)


def _combine_body(c0_ref, c1_ref, g0_ref, g1_ref, y_ref):
    comb = g0_ref[...] * c0_ref[...] + g1_ref[...] * c1_ref[...]
    eps = np.float32(np.finfo(np.float64).eps)
    comb = jnp.where(comb == 0.0, eps, comb)
    y_ref[...] = jnp.log(comb)


def _combine(c0, c1, g0, g1):
    cblk = 512
    row = pl.BlockSpec((cblk, D), lambda i: (i, 0))
    gsp = pl.BlockSpec((cblk, 1), lambda i: (i, 0))
    return pl.pallas_call(
        _combine_body,
        grid=(B // cblk,),
        in_specs=[row, row, gsp, gsp],
        out_specs=row,
        out_shape=jax.ShapeDtypeStruct((B, D), jnp.float32),
    )(c0, c1, g0, g1)


@jax.jit
def kernel(x, w_gate, Wq, Wk, Wv, Wo):
    s0, s1, g0, g1, tab = _routing(x, w_gate)
    tab1d = tab.reshape(128)
    s0f = s0.reshape(B)
    s1f = s1.reshape(B)
    xg = _sc_scatter(x, s0f, s1f)
    cg = _mega(tab1d, xg, Wq, Wk, Wv, Wo)
    c0, c1 = _sc_gather(cg, s0f, s1f)
    return _combine(c0, c1, g0, g1)
